# baseline (speedup 1.0000x reference)
import functools

import jax
import jax.numpy as jnp
from jax import lax
from jax.experimental import pallas as pl
from jax.experimental.pallas import tpu as pltpu
from jax.experimental.pallas import tpu_sc as plsc

LANES = 16


def _table_kernel(w_ref, b_ref, emb_ref, out_ref, *, inv_seq):
    t = jnp.dot(emb_ref[...], w_ref[...], preferred_element_type=jnp.float32)
    t = (t + b_ref[...].reshape(1, -1)) * inv_seq
    u = lax.bitcast_convert_type(t, jnp.uint32) + jnp.uint32(0x8000)
    hi = u[:, 0:1] & jnp.uint32(0xFFFF0000)
    lo = u[:, 1:2] >> 16
    out_ref[...] = lax.bitcast_convert_type(hi | lo, jnp.int32)


def _build_table(emb, W, b, seq):
    vocab, d = emb.shape
    c = W.shape[1]
    blk = 4000
    assert vocab % blk == 0
    return pl.pallas_call(
        functools.partial(_table_kernel, inv_seq=1.0 / seq),
        grid=(vocab // blk,),
        in_specs=[
            pl.BlockSpec((d, c), lambda i: (0, 0)),
            pl.BlockSpec((c,), lambda i: (0,)),
            pl.BlockSpec((blk, d), lambda i: (i, 0)),
        ],
        out_specs=pl.BlockSpec((blk, 1), lambda i: (i, 0)),
        out_shape=jax.ShapeDtypeStruct((vocab, 1), jnp.int32),
    )(W, b, emb)


def _make_sc_pool(vocab, batch, seq):
    nc, ns = 2, 16
    nw = nc * ns
    rows_per_tile = batch // nw
    groups = rows_per_tile // LANES

    mesh = plsc.VectorSubcoreMesh(
        core_axis_name="c", subcore_axis_name="s",
        num_cores=nc, num_subcores=ns)

    @functools.partial(
        pl.kernel,
        mesh=mesh,
        out_type=jax.ShapeDtypeStruct((nc, batch), jnp.float32),
        scratch_types=[
            pltpu.VMEM((vocab,), jnp.int32),
            pltpu.VMEM((LANES, seq), jnp.int32),
            pltpu.VMEM((LANES, seq), jnp.int32),
            pltpu.VMEM((rows_per_tile,), jnp.float32),
            pltpu.VMEM((rows_per_tile,), jnp.float32),
            pltpu.SemaphoreType.DMA,
            pltpu.SemaphoreType.DMA,
            pltpu.SemaphoreType.DMA,
        ],
        compiler_params=pltpu.CompilerParams(
            use_tc_tiling_on_sc=False, needs_layout_passes=False),
    )
    def pool(tab_hbm, x_hbm, out_hbm, tab_v, x_v0, x_v1, out_v0, out_v1,
             tab_sem, sem0, sem1):
        cid = lax.axis_index("c")
        sid = lax.axis_index("s")
        wid = cid * ns + sid
        base_row = wid * rows_per_tile
        x_bufs = (x_v0, x_v1)
        x_sems = (sem0, sem1)

        def x_copy(g, buf):
            return pltpu.async_copy(
                x_hbm.at[pl.ds(base_row + g * LANES, LANES)],
                x_bufs[buf], x_sems[buf])

        tab_cp = pltpu.async_copy(tab_hbm, tab_v, tab_sem)
        cps = [x_copy(0, 0), x_copy(1, 1)]
        tab_cp.wait()

        himask = jnp.full((LANES,), -0x10000, jnp.int32)
        riota = lax.iota(jnp.int32, LANES)
        for g in range(groups):
            buf = g % 2
            cps[buf].wait()

            def step(i, accs):
                new = []
                for k in range(2):
                    col = jnp.full((LANES,), i * 2 + k, jnp.int32)
                    iv = plsc.load_gather(x_bufs[buf], [riota, col])
                    pv = plsc.load_gather(tab_v, [iv])
                    v0 = plsc.bitcast(pv & himask, jnp.float32)
                    v1 = plsc.bitcast(pv << 16, jnp.float32)
                    new.append((accs[k][0] + v0, accs[k][1] + v1))
                return tuple(new)

            zero = jnp.zeros((LANES,), jnp.float32)
            (a00, a01), (a10, a11) = lax.fori_loop(
                0, seq // 2, step, ((zero, zero), (zero, zero)), unroll=4)
            out_v0[pl.ds(g * LANES, LANES)] = a00 + a10
            out_v1[pl.ds(g * LANES, LANES)] = a01 + a11
            if g + 2 < groups:
                cps[buf] = x_copy(g + 2, buf)

        pltpu.sync_copy(out_v0, out_hbm.at[0, pl.ds(base_row, rows_per_tile)])
        pltpu.sync_copy(out_v1, out_hbm.at[1, pl.ds(base_row, rows_per_tile)])

    return pool


def kernel(x, emb, W, b):
    batch, seq = x.shape
    tab = _build_table(emb, W, b, seq).reshape(-1)
    pool = _make_sc_pool(tab.shape[0], batch, seq)
    return pool(tab, x.astype(jnp.int32)).T


# speedup vs baseline: 1.5396x; 1.5396x over previous
import functools

import jax
import jax.numpy as jnp
from jax import lax
from jax.experimental import pallas as pl
from jax.experimental.pallas import tpu as pltpu
from jax.experimental.pallas import tpu_sc as plsc

LANES = 16


def _table_kernel(w_ref, b_ref, emb_ref, out_ref, *, inv_seq):
    t = lax.dot_general(
        w_ref[...], emb_ref[...],
        dimension_numbers=(((0,), (1,)), ((), ())),
        preferred_element_type=jnp.float32,
    )
    t = (t + b_ref[...].reshape(-1, 1)) * inv_seq
    u = lax.bitcast_convert_type(t, jnp.uint32) + jnp.uint32(0x8000)
    hi = u[0:1] & jnp.uint32(0xFFFF0000)
    lo = u[1:2] >> 16
    out_ref[...] = lax.bitcast_convert_type(hi | lo, jnp.int32)


def _build_table(emb, W, b, seq):
    vocab, d = emb.shape
    c = W.shape[1]
    blk = 8192
    grid = pl.cdiv(vocab, blk)
    vpad = grid * blk
    return pl.pallas_call(
        functools.partial(_table_kernel, inv_seq=1.0 / seq),
        grid=(grid,),
        in_specs=[
            pl.BlockSpec((d, c), lambda i: (0, 0)),
            pl.BlockSpec((c,), lambda i: (0,)),
            pl.BlockSpec((blk, d), lambda i: (i, 0)),
        ],
        out_specs=pl.BlockSpec((1, blk), lambda i: (0, i)),
        out_shape=jax.ShapeDtypeStruct((1, vpad), jnp.int32),
    )(W, b, emb)


def _make_sc_pool(vpad, batch, seq):
    nc, ns = 2, 16
    nw = nc * ns
    rows_per_tile = batch // nw
    groups = rows_per_tile // LANES

    mesh = plsc.VectorSubcoreMesh(
        core_axis_name="c", subcore_axis_name="s",
        num_cores=nc, num_subcores=ns)

    @functools.partial(
        pl.kernel,
        mesh=mesh,
        out_type=jax.ShapeDtypeStruct((nc, batch), jnp.float32),
        scratch_types=[
            pltpu.VMEM((vpad,), jnp.int32),
            pltpu.VMEM((LANES, seq), jnp.int32),
            pltpu.VMEM((LANES, seq), jnp.int32),
            pltpu.VMEM((rows_per_tile,), jnp.float32),
            pltpu.VMEM((rows_per_tile,), jnp.float32),
            pltpu.SemaphoreType.DMA,
            pltpu.SemaphoreType.DMA,
            pltpu.SemaphoreType.DMA,
        ],
        compiler_params=pltpu.CompilerParams(
            use_tc_tiling_on_sc=True, needs_layout_passes=False),
    )
    def pool(tab_hbm, x_hbm, out_hbm, tab_v, x_v0, x_v1, out_v0, out_v1,
             tab_sem, sem0, sem1):
        cid = lax.axis_index("c")
        sid = lax.axis_index("s")
        wid = cid * ns + sid
        base_row = wid * rows_per_tile
        x_bufs = (x_v0, x_v1)
        x_sems = (sem0, sem1)

        def x_copy(g, buf):
            return pltpu.async_copy(
                x_hbm.at[pl.ds(base_row + g * LANES, LANES)],
                x_bufs[buf], x_sems[buf])

        tab_cp = pltpu.async_copy(tab_hbm.at[0], tab_v, tab_sem)
        cps = [x_copy(0, 0), x_copy(1, 1)]
        tab_cp.wait()

        himask = jnp.full((LANES,), -0x10000, jnp.int32)
        riota = lax.iota(jnp.int32, LANES)
        for g in range(groups):
            buf = g % 2
            cps[buf].wait()

            def step(i, accs):
                new = []
                for k in range(2):
                    col = jnp.full((LANES,), i * 2 + k, jnp.int32)
                    iv = plsc.load_gather(x_bufs[buf], [riota, col])
                    pv = plsc.load_gather(tab_v, [iv])
                    v0 = plsc.bitcast(pv & himask, jnp.float32)
                    v1 = plsc.bitcast(pv << 16, jnp.float32)
                    new.append((accs[k][0] + v0, accs[k][1] + v1))
                return tuple(new)

            zero = jnp.zeros((LANES,), jnp.float32)
            (a00, a01), (a10, a11) = lax.fori_loop(
                0, seq // 2, step, ((zero, zero), (zero, zero)), unroll=4)
            out_v0[pl.ds(g * LANES, LANES)] = a00 + a10
            out_v1[pl.ds(g * LANES, LANES)] = a01 + a11
            if g + 2 < groups:
                cps[buf] = x_copy(g + 2, buf)

        pltpu.sync_copy(out_v0, out_hbm.at[0, pl.ds(base_row, rows_per_tile)])
        pltpu.sync_copy(out_v1, out_hbm.at[1, pl.ds(base_row, rows_per_tile)])

    return pool


def kernel(x, emb, W, b):
    batch, seq = x.shape
    tab = _build_table(emb, W, b, seq)
    pool = _make_sc_pool(tab.shape[1], batch, seq)
    return pool(tab, x.astype(jnp.int32)).T


# speedup vs baseline: 1.7718x; 1.1508x over previous
import functools

import jax
import jax.numpy as jnp
from jax import lax
from jax.experimental import pallas as pl
from jax.experimental.pallas import tpu as pltpu
from jax.experimental.pallas import tpu_sc as plsc

LANES = 16


def _table_kernel(w_ref, b_ref, emb_ref, out_ref, *, inv_seq):
    t = lax.dot_general(
        w_ref[...], emb_ref[...],
        dimension_numbers=(((0,), (1,)), ((), ())),
        preferred_element_type=jnp.float32,
    )
    t = (t + b_ref[...].reshape(-1, 1)) * inv_seq
    u = lax.bitcast_convert_type(t, jnp.uint32) + jnp.uint32(0x8000)
    hi = u[0:1] & jnp.uint32(0xFFFF0000)
    lo = u[1:2] >> 16
    out_ref[...] = lax.bitcast_convert_type(hi | lo, jnp.int32)


def _build_table(emb, W, b, seq):
    vocab, d = emb.shape
    c = W.shape[1]
    blk = 8192
    grid = pl.cdiv(vocab, blk)
    vpad = grid * blk
    return pl.pallas_call(
        functools.partial(_table_kernel, inv_seq=1.0 / seq),
        grid=(grid,),
        in_specs=[
            pl.BlockSpec((d, c), lambda i: (0, 0)),
            pl.BlockSpec((c,), lambda i: (0,)),
            pl.BlockSpec((blk, d), lambda i: (i, 0)),
        ],
        out_specs=pl.BlockSpec((1, blk), lambda i: (0, i)),
        out_shape=jax.ShapeDtypeStruct((1, vpad), jnp.int32),
    )(W, b, emb)


def _make_sc_pool(vpad, batch, seq):
    nc, ns = 2, 16
    nw = nc * ns
    rows_per_tile = batch // nw
    groups = rows_per_tile // LANES

    mesh = plsc.VectorSubcoreMesh(
        core_axis_name="c", subcore_axis_name="s",
        num_cores=nc, num_subcores=ns)

    @functools.partial(
        pl.kernel,
        mesh=mesh,
        out_type=jax.ShapeDtypeStruct((nc, batch), jnp.float32),
        scratch_types=[
            pltpu.VMEM((vpad,), jnp.int32),
            pltpu.VMEM_SHARED((vpad,), jnp.int32),
            pltpu.VMEM((LANES, seq), jnp.int32),
            pltpu.VMEM((LANES, seq), jnp.int32),
            pltpu.VMEM((rows_per_tile,), jnp.float32),
            pltpu.VMEM((rows_per_tile,), jnp.float32),
            pltpu.SemaphoreType.DMA,
            pltpu.SemaphoreType.DMA,
            pltpu.SemaphoreType.DMA,
        ],
        compiler_params=pltpu.CompilerParams(
            use_tc_tiling_on_sc=False, needs_layout_passes=False),
    )
    def pool(tab_hbm, x_hbm, out_hbm, tab_v, tab_sh, x_v0, x_v1,
             out_v0, out_v1, tab_sem, sem0, sem1):
        cid = lax.axis_index("c")
        sid = lax.axis_index("s")
        wid = cid * ns + sid
        base_row = wid * rows_per_tile
        x_bufs = (x_v0, x_v1)
        x_sems = (sem0, sem1)

        def x_copy(g, buf):
            return pltpu.async_copy(
                x_hbm.at[pl.ds(base_row + g * LANES, LANES)],
                x_bufs[buf], x_sems[buf])

        @pl.when(sid == 0)
        def _():
            pltpu.sync_copy(tab_hbm.at[0], tab_sh)

        cps = [x_copy(0, 0), x_copy(1, 1)]
        plsc.subcore_barrier()
        pltpu.async_copy(tab_sh, tab_v, tab_sem).wait()

        himask = jnp.full((LANES,), -0x10000, jnp.int32)
        riota = lax.iota(jnp.int32, LANES)
        for g in range(groups):
            buf = g % 2
            cps[buf].wait()

            def step(i, accs):
                new = []
                for k in range(2):
                    col = jnp.full((LANES,), i * 2 + k, jnp.int32)
                    iv = plsc.load_gather(x_bufs[buf], [riota, col])
                    pv = plsc.load_gather(tab_v, [iv])
                    v0 = plsc.bitcast(pv & himask, jnp.float32)
                    v1 = plsc.bitcast(pv << 16, jnp.float32)
                    new.append((accs[k][0] + v0, accs[k][1] + v1))
                return tuple(new)

            zero = jnp.zeros((LANES,), jnp.float32)
            (a00, a01), (a10, a11) = lax.fori_loop(
                0, seq // 2, step, ((zero, zero), (zero, zero)), unroll=4)
            out_v0[pl.ds(g * LANES, LANES)] = a00 + a10
            out_v1[pl.ds(g * LANES, LANES)] = a01 + a11
            if g + 2 < groups:
                cps[buf] = x_copy(g + 2, buf)

        pltpu.sync_copy(out_v0, out_hbm.at[0, pl.ds(base_row, rows_per_tile)])
        pltpu.sync_copy(out_v1, out_hbm.at[1, pl.ds(base_row, rows_per_tile)])

    return pool


def kernel(x, emb, W, b):
    batch, seq = x.shape
    tab = _build_table(emb, W, b, seq)
    pool = _make_sc_pool(tab.shape[1], batch, seq)
    return pool(tab, x.astype(jnp.int32)).T


# speedup vs baseline: 1.8254x; 1.0302x over previous
import functools

import jax
import jax.numpy as jnp
from jax import lax
from jax.experimental import pallas as pl
from jax.experimental.pallas import tpu as pltpu
from jax.experimental.pallas import tpu_sc as plsc

LANES = 16


def _table_kernel(w_ref, b_ref, emb_ref, out_ref, *, inv_seq):
    t = lax.dot_general(
        w_ref[...], emb_ref[...],
        dimension_numbers=(((0,), (1,)), ((), ())),
        preferred_element_type=jnp.float32,
    )
    t = (t + b_ref[...].reshape(-1, 1)) * inv_seq
    u = lax.bitcast_convert_type(t, jnp.uint32) + jnp.uint32(0x8000)
    hi = u[0:1] & jnp.uint32(0xFFFF0000)
    lo = u[1:2] >> 16
    out_ref[...] = lax.bitcast_convert_type(hi | lo, jnp.int32)


def _build_table(emb, W, b, seq):
    vocab, d = emb.shape
    c = W.shape[1]
    blk = 16384
    grid = pl.cdiv(vocab, blk)
    vpad = grid * blk
    return pl.pallas_call(
        functools.partial(_table_kernel, inv_seq=1.0 / seq),
        grid=(grid,),
        in_specs=[
            pl.BlockSpec((d, c), lambda i: (0, 0)),
            pl.BlockSpec((c,), lambda i: (0,)),
            pl.BlockSpec((blk, d), lambda i: (i, 0)),
        ],
        out_specs=pl.BlockSpec((1, blk), lambda i: (0, i)),
        out_shape=jax.ShapeDtypeStruct((1, vpad), jnp.int32),
    )(W, b, emb)


def _make_sc_pool(vpad, batch, seq):
    nc, ns = 2, 16
    nw = nc * ns
    rows_per_tile = batch // nw
    groups = rows_per_tile // LANES

    mesh = plsc.VectorSubcoreMesh(
        core_axis_name="c", subcore_axis_name="s",
        num_cores=nc, num_subcores=ns)

    @functools.partial(
        pl.kernel,
        mesh=mesh,
        out_type=jax.ShapeDtypeStruct((nc, batch), jnp.float32),
        scratch_types=[
            pltpu.VMEM((vpad,), jnp.int32),
            pltpu.VMEM_SHARED((vpad,), jnp.int32),
            pltpu.VMEM((LANES, seq), jnp.int32),
            pltpu.VMEM((LANES, seq), jnp.int32),
            pltpu.VMEM((rows_per_tile,), jnp.float32),
            pltpu.VMEM((rows_per_tile,), jnp.float32),
            pltpu.SemaphoreType.DMA,
            pltpu.SemaphoreType.DMA,
            pltpu.SemaphoreType.DMA,
        ],
        compiler_params=pltpu.CompilerParams(
            use_tc_tiling_on_sc=False, needs_layout_passes=False),
    )
    def pool(tab_hbm, x_hbm, out_hbm, tab_v, tab_sh, x_v0, x_v1,
             out_v0, out_v1, tab_sem, sem0, sem1):
        cid = lax.axis_index("c")
        sid = lax.axis_index("s")
        wid = cid * ns + sid
        base_row = wid * rows_per_tile
        x_bufs = (x_v0, x_v1)
        x_sems = (sem0, sem1)

        def x_copy(g, buf):
            return pltpu.async_copy(
                x_hbm.at[pl.ds(base_row + g * LANES, LANES)],
                x_bufs[buf], x_sems[buf])

        @pl.when(sid == 0)
        def _():
            pltpu.sync_copy(tab_hbm.at[0], tab_sh)

        cps = [x_copy(0, 0), x_copy(1, 1)]
        plsc.subcore_barrier()
        pltpu.async_copy(tab_sh, tab_v, tab_sem).wait()

        himask = jnp.full((LANES,), -0x10000, jnp.int32)
        riota = lax.iota(jnp.int32, LANES)
        for g in range(groups):
            buf = g % 2
            cps[buf].wait()

            def step(i, accs):
                new = []
                for k in range(2):
                    col = jnp.full((LANES,), i * 2 + k, jnp.int32)
                    iv = plsc.load_gather(x_bufs[buf], [riota, col])
                    pv = plsc.load_gather(tab_v, [iv])
                    v0 = plsc.bitcast(pv & himask, jnp.float32)
                    v1 = plsc.bitcast(pv << 16, jnp.float32)
                    new.append((accs[k][0] + v0, accs[k][1] + v1))
                return tuple(new)

            zero = jnp.zeros((LANES,), jnp.float32)
            (a00, a01), (a10, a11) = lax.fori_loop(
                0, seq // 2, step, ((zero, zero), (zero, zero)), unroll=4)
            out_v0[pl.ds(g * LANES, LANES)] = a00 + a10
            out_v1[pl.ds(g * LANES, LANES)] = a01 + a11
            if g + 2 < groups:
                cps[buf] = x_copy(g + 2, buf)

        pltpu.sync_copy(out_v0, out_hbm.at[0, pl.ds(base_row, rows_per_tile)])
        pltpu.sync_copy(out_v1, out_hbm.at[1, pl.ds(base_row, rows_per_tile)])

    return pool


def kernel(x, emb, W, b):
    batch, seq = x.shape
    tab = _build_table(emb, W, b, seq)
    pool = _make_sc_pool(tab.shape[1], batch, seq)
    return pool(tab, x.astype(jnp.int32)).T


# speedup vs baseline: 1.8389x; 1.0074x over previous
import functools

import jax
import jax.numpy as jnp
from jax import lax
from jax.experimental import pallas as pl
from jax.experimental.pallas import tpu as pltpu
from jax.experimental.pallas import tpu_sc as plsc

LANES = 16


def _table_kernel(w_ref, b_ref, emb_ref, out_ref, *, inv_seq):
    t = lax.dot_general(
        w_ref[...], emb_ref[...],
        dimension_numbers=(((0,), (1,)), ((), ())),
        preferred_element_type=jnp.float32,
    )
    t = (t + b_ref[...].reshape(-1, 1)) * inv_seq
    u = lax.bitcast_convert_type(t, jnp.uint32) + jnp.uint32(0x8000)
    hi = u[0:1] & jnp.uint32(0xFFFF0000)
    lo = u[1:2] >> 16
    out_ref[...] = lax.bitcast_convert_type(hi | lo, jnp.int32)


def _build_table(emb, W, b, seq):
    vocab, d = emb.shape
    c = W.shape[1]
    blk = 25600
    grid = pl.cdiv(vocab, blk)
    vpad = grid * blk
    return pl.pallas_call(
        functools.partial(_table_kernel, inv_seq=1.0 / seq),
        grid=(grid,),
        in_specs=[
            pl.BlockSpec((d, c), lambda i: (0, 0)),
            pl.BlockSpec((c,), lambda i: (0,)),
            pl.BlockSpec((blk, d), lambda i: (i, 0)),
        ],
        out_specs=pl.BlockSpec((1, blk), lambda i: (0, i)),
        out_shape=jax.ShapeDtypeStruct((1, vpad), jnp.int32),
    )(W, b, emb)


def _make_sc_pool(vpad, batch, seq):
    nc, ns = 2, 16
    nw = nc * ns
    rows_per_tile = batch // nw
    groups = rows_per_tile // LANES

    mesh = plsc.VectorSubcoreMesh(
        core_axis_name="c", subcore_axis_name="s",
        num_cores=nc, num_subcores=ns)

    @functools.partial(
        pl.kernel,
        mesh=mesh,
        out_type=jax.ShapeDtypeStruct((nc, batch), jnp.float32),
        scratch_types=[
            pltpu.VMEM((vpad,), jnp.int32),
            pltpu.VMEM_SHARED((vpad,), jnp.int32),
            pltpu.VMEM((LANES, seq), jnp.int32),
            pltpu.VMEM((LANES, seq), jnp.int32),
            pltpu.VMEM((rows_per_tile,), jnp.float32),
            pltpu.VMEM((rows_per_tile,), jnp.float32),
            pltpu.SemaphoreType.DMA,
            pltpu.SemaphoreType.DMA,
            pltpu.SemaphoreType.DMA,
        ],
        compiler_params=pltpu.CompilerParams(
            use_tc_tiling_on_sc=False, needs_layout_passes=False),
    )
    def pool(tab_hbm, x_hbm, out_hbm, tab_v, tab_sh, x_v0, x_v1,
             out_v0, out_v1, tab_sem, sem0, sem1):
        cid = lax.axis_index("c")
        sid = lax.axis_index("s")
        wid = cid * ns + sid
        base_row = wid * rows_per_tile
        x_bufs = (x_v0, x_v1)
        x_sems = (sem0, sem1)

        def x_copy(g, buf):
            return pltpu.async_copy(
                x_hbm.at[pl.ds(base_row + g * LANES, LANES)],
                x_bufs[buf], x_sems[buf])

        @pl.when(sid == 0)
        def _():
            pltpu.sync_copy(tab_hbm.at[0], tab_sh)

        cps = [x_copy(0, 0), x_copy(1, 1)]
        plsc.subcore_barrier()
        pltpu.async_copy(tab_sh, tab_v, tab_sem).wait()

        himask = jnp.full((LANES,), -0x10000, jnp.int32)
        riota = lax.iota(jnp.int32, LANES)
        for g in range(groups):
            buf = g % 2
            cps[buf].wait()

            def step(i, accs):
                new = []
                for k in range(2):
                    col = jnp.full((LANES,), i * 2 + k, jnp.int32)
                    iv = plsc.load_gather(x_bufs[buf], [riota, col])
                    pv = plsc.load_gather(tab_v, [iv])
                    v0 = plsc.bitcast(pv & himask, jnp.float32)
                    v1 = plsc.bitcast(pv << 16, jnp.float32)
                    new.append((accs[k][0] + v0, accs[k][1] + v1))
                return tuple(new)

            zero = jnp.zeros((LANES,), jnp.float32)
            (a00, a01), (a10, a11) = lax.fori_loop(
                0, seq // 2, step, ((zero, zero), (zero, zero)), unroll=4)
            out_v0[pl.ds(g * LANES, LANES)] = a00 + a10
            out_v1[pl.ds(g * LANES, LANES)] = a01 + a11
            if g + 2 < groups:
                cps[buf] = x_copy(g + 2, buf)

        pltpu.sync_copy(out_v0, out_hbm.at[0, pl.ds(base_row, rows_per_tile)])
        pltpu.sync_copy(out_v1, out_hbm.at[1, pl.ds(base_row, rows_per_tile)])

    return pool


def kernel(x, emb, W, b):
    batch, seq = x.shape
    tab = _build_table(emb, W, b, seq)
    pool = _make_sc_pool(tab.shape[1], batch, seq)
    return pool(tab, x.astype(jnp.int32)).T
